# baseline (speedup 1.0000x reference)
import functools

import jax
import jax.numpy as jnp
from jax import lax
from jax.experimental import pallas as pl
from jax.experimental.pallas import tpu as pltpu
from jax.experimental.pallas import tpu_sc as plsc

_D = 64
_SEQ = 200
_BATCH = 4096
_NC = 2
_NS = 16
_NW = _NC * _NS
_CH = 128


def _make_gather():
    mesh = plsc.VectorSubcoreMesh(core_axis_name="c", subcore_axis_name="s")

    @functools.partial(
        pl.kernel,
        mesh=mesh,
        out_type=jax.ShapeDtypeStruct((_SEQ, _D, _BATCH), jnp.float32),
        scratch_types=[
            pltpu.VMEM((_SEQ, _CH), jnp.int32),
            pltpu.VMEM((2, _CH), jnp.int32),
            pltpu.VMEM((2, _CH, 128), jnp.float32),
            pltpu.VMEM((2, _D, _CH), jnp.float32),
            pltpu.SemaphoreType.DMA,
            pltpu.SemaphoreType.DMA,
        ],
        compiler_params=pltpu.CompilerParams(
            use_tc_tiling_on_sc=True, needs_layout_passes=False
        ),
    )
    def emb(xT_hbm, tbl2_hbm, out_hbm, idx_v, u_v, g_v, t_v, gsem, osem):
        wid = lax.axis_index("s") * _NC + lax.axis_index("c")
        iota = lax.iota(jnp.int32, 16)
        pltpu.sync_copy(xT_hbm.at[:, pl.ds(wid * _CH, _CH)], idx_v)

        def fire_gather(st, gi):
            for r in range(8):
                u_v.at[st][pl.ds(r * 16, 16)] = lax.shift_right_logical(
                    idx_v.at[gi][pl.ds(r * 16, 16)], 1
                )
            pltpu.async_copy(tbl2_hbm.at[u_v.at[st]], g_v.at[st], gsem)

        def drain_gather(st):
            pltpu.make_async_copy(tbl2_hbm.at[u_v.at[st]],
                                  g_v.at[st], gsem).wait()

        def transpose(st, gi):
            rows = [bv * 16 + iota for bv in range(8)]
            half = [lax.bitwise_and(idx_v.at[gi][pl.ds(bv * 16, 16)], 1) * _D
                    for bv in range(8)]
            for bv in range(8):
                for dg in range(_D // 8):
                    vals = [plsc.load_gather(
                        g_v.at[st], [rows[bv], half[bv] + (dg * 8 + k)])
                        for k in range(8)]
                    for k in range(8):
                        t_v.at[st].at[dg * 8 + k][pl.ds(bv * 16, 16)] = vals[k]

        def fire_out(st, gi):
            pltpu.async_copy(t_v.at[st],
                             out_hbm.at[gi, :, pl.ds(wid * _CH, _CH)], osem)

        def drain_out(st):
            pltpu.make_async_copy(t_v.at[st],
                                  out_hbm.at[0, :, pl.ds(0, _CH)], osem).wait()

        fire_gather(0, 0)
        fire_gather(1, 1)

        def body(p, carry):
            for st in range(2):
                gi = 2 * p + st
                drain_gather(st)
                pl.when(p > 0)(lambda: drain_out(st))
                transpose(st, gi)
                fire_out(st, gi)
                fire_gather(st, jnp.minimum(gi + 2, _SEQ - 1))
            return carry

        lax.fori_loop(0, _SEQ // 2, body, 0)
        drain_gather(0)
        drain_gather(1)
        drain_out(0)
        drain_out(1)

    return emb


_emb_gather = _make_gather()

_VOC = 1000000
_NVT = (_VOC + 127) // 128
_BPW = (_NVT + _NW - 1) // _NW


def _make_transpose():
    mesh = plsc.VectorSubcoreMesh(core_axis_name="c", subcore_axis_name="s")

    @functools.partial(
        pl.kernel,
        mesh=mesh,
        out_type=jax.ShapeDtypeStruct((_VOC // 2, 128), jnp.float32),
        scratch_types=[
            pltpu.VMEM((2, _D, 128), jnp.float32),
            pltpu.VMEM((2, _D, 128), jnp.float32),
            pltpu.SemaphoreType.DMA,
            pltpu.SemaphoreType.DMA,
        ],
        compiler_params=pltpu.CompilerParams(
            use_tc_tiling_on_sc=True, needs_layout_passes=False
        ),
    )
    def tr(tblT_hbm, tail_hbm, tbl2_hbm, a_v, b_v, isem, osem):
        wid = lax.axis_index("s") * _NC + lax.axis_index("c")
        iota = lax.iota(jnp.int32, 16)

        def voff_of(k):
            vt = jnp.minimum(wid + _NW * k, _NVT - 2)
            return vt * 128

        def fire_in(st, k):
            pltpu.async_copy(tblT_hbm.at[:, pl.ds(voff_of(k), 128)],
                             a_v.at[st], isem)

        def drain_in(st):
            pltpu.make_async_copy(tblT_hbm.at[:, pl.ds(0, 128)],
                                  a_v.at[st], isem).wait()

        def transpose(st):
            for cv in range(8):
                rows = (cv % 4) * 16 + iota
                cbase = iota * 0 + (cv // 4)
                for ug in range(8):
                    vals = [plsc.load_gather(
                        a_v.at[st], [rows, cbase + 2 * (ug * 8 + j)])
                        for j in range(8)]
                    for j in range(8):
                        b_v.at[st].at[ug * 8 + j][pl.ds(cv * 16, 16)] = vals[j]

        def fire_out(st, k):
            uoff = pl.multiple_of(
                lax.shift_right_logical(voff_of(k), 1), _D)
            pltpu.async_copy(b_v.at[st], tbl2_hbm.at[pl.ds(uoff, _D)], osem)

        def drain_out(st):
            pltpu.make_async_copy(b_v.at[st],
                                  tbl2_hbm.at[pl.ds(0, _D)], osem).wait()

        fire_in(0, 0)
        fire_in(1, 1)

        def body(p, carry):
            for st in range(2):
                k = 2 * p + st
                drain_in(st)
                pl.when(p > 0)(lambda: drain_out(st))
                transpose(st)
                fire_out(st, k)
                fire_in(st, jnp.minimum(k + 2, _BPW - 1))
            return carry

        lax.fori_loop(0, (_BPW + 1) // 2, body, 0)
        drain_in(0)
        drain_in(1)
        drain_out(0)
        drain_out(1)

        def tail():
            pltpu.sync_copy(tail_hbm, b_v.at[0].at[pl.ds(0, 32)])
            pltpu.sync_copy(b_v.at[0].at[pl.ds(0, 32)],
                            tbl2_hbm.at[pl.ds((_NVT - 1) * 64, 32)])

        pl.when(wid == 0)(tail)

    return tr


_tbl_transpose = _make_transpose()


def kernel(x, table):
    xT = x.T
    tail32 = table[(_NVT - 1) * 128:].reshape(32, 128)
    tbl2 = _tbl_transpose(table.T, tail32)
    outT = _emb_gather(xT, tbl2)
    return jnp.transpose(outT, (2, 0, 1))


# speedup vs baseline: 1.3525x; 1.3525x over previous
import functools

import jax
import jax.numpy as jnp
from jax import lax
from jax.experimental import pallas as pl
from jax.experimental.pallas import tpu as pltpu
from jax.experimental.pallas import tpu_sc as plsc

_D = 64
_B = 4096 * 200
_NC = 2
_NS = 16
_NW = _NC * _NS
_CH = 128
_ROWS_PER_W = _B // _NW
_NCH = _ROWS_PER_W // _CH
_K = 4
_NG = _NCH // _K


def _make_emb():
    mesh = plsc.VectorSubcoreMesh(core_axis_name="c", subcore_axis_name="s")

    @functools.partial(
        pl.kernel,
        mesh=mesh,
        out_type=jax.ShapeDtypeStruct((_B, _D), jnp.float32),
        scratch_types=[
            pltpu.VMEM((_NCH, _CH), jnp.int32),
            pltpu.VMEM((2, _K, _CH, _D), jnp.float32),
            pltpu.SemaphoreType.DMA,
            pltpu.SemaphoreType.DMA,
        ],
        compiler_params=pltpu.CompilerParams(use_tc_tiling_on_sc=False),
    )
    def emb(idx_hbm, table_hbm, out_hbm, idx_v, rows_v, gsem, osem):
        wid = lax.axis_index("s") * _NC + lax.axis_index("c")
        pltpu.sync_copy(idx_hbm.at[pl.ds(wid * _NCH, _NCH)], idx_v)
        base = wid * _ROWS_PER_W

        def fire_g(s, g):
            for b in range(_K):
                pltpu.async_copy(table_hbm.at[idx_v.at[g * _K + b]],
                                 rows_v.at[s, b], gsem)

        def drain_g(s):
            for b in range(_K):
                pltpu.make_async_copy(table_hbm.at[idx_v.at[0]],
                                      rows_v.at[s, b], gsem).wait()

        def fire_o(s, g):
            for b in range(_K):
                pltpu.async_copy(rows_v.at[s, b],
                                 out_hbm.at[pl.ds(base + (g * _K + b) * _CH, _CH)],
                                 osem)

        def drain_o(s):
            for b in range(_K):
                pltpu.make_async_copy(rows_v.at[s, b],
                                      out_hbm.at[pl.ds(base, _CH)], osem).wait()

        fire_g(0, 0)
        fire_g(1, 1)
        drain_g(0)
        fire_o(0, 0)
        drain_g(1)
        fire_o(1, 1)

        def body(p, carry):
            for s in range(2):
                g = 2 * p + s
                drain_o(s)
                fire_g(s, g)
                drain_g(s)
                fire_o(s, g)
            return carry

        lax.fori_loop(1, _NG // 2, body, 0)
        drain_o(0)
        drain_o(1)

    return emb


_emb = _make_emb()


def kernel(x, table):
    idx = x.reshape(_B // _CH, _CH)
    out = _emb(idx, table)
    return out.reshape(x.shape[0], x.shape[1], _D)
